# Initial kernel scaffold; baseline (speedup 1.0000x reference)
import jax
import jax.numpy as jnp
from jax.experimental import pallas as pl


def kernel(x, codebooks):
    raise NotImplementedError("write your pallas kernel here")


# speedup vs baseline: 1.5462x; 1.5462x over previous
import jax
import jax.numpy as jnp
from jax.experimental import pallas as pl
from jax.experimental.pallas import tpu as pltpu

_B, _N, _D, _K, _Q = 8, 576, 256, 1024, 4
_M = _B * _N
_T = 512


def _rowsum_t(a):
    t = a[:, :128] + a[:, 128:]
    tt = t.T
    acc = tt[0:8, :]
    for c in range(1, 16):
        acc = acc + tt[8 * c:8 * (c + 1), :]
    acc = acc[0:4, :] + acc[4:8, :]
    acc = acc[0:2, :] + acc[2:4, :]
    return acc[0:1, :] + acc[1:2, :]


def _rowsum(a):
    return _rowsum_t(a).T


def _rvq_body(x_ref, cb_ref, hi_ref, mid_ref, lo_ref, qout_ref, idx_ref,
              loss_ref, e2_ref, sc1_ref, sc2_ref):
    step = pl.program_id(0)

    @pl.when(step == 0)
    def _prep():
        for qi in range(_Q):
            emb = cb_ref[qi]
            e2_ref[qi:qi + 1, :] = _rowsum_t(emb * emb)
        loss_ref[...] = jnp.zeros_like(loss_ref)

    eps = 1e-6
    resid = x_ref[...]
    qout = jnp.zeros_like(resid)
    iota = jax.lax.broadcasted_iota(jnp.int32, (_T, _K), 1)
    iota_f = iota.astype(jnp.float32)
    idx_cols = []
    loss_cols = []
    for qi in range(_Q):
        hi = hi_ref[qi]
        mid = mid_ref[qi]
        lo = lo_ref[qi]
        lhs = (2.0 * resid).astype(jnp.bfloat16)
        cross2 = jax.lax.dot_general(
            lhs, hi, (((1,), (1,)), ((), ())), preferred_element_type=jnp.float32
        )
        r2 = _rowsum(resid * resid)
        dist = (r2 - cross2) + e2_ref[qi:qi + 1, :]
        mval = jnp.min(dist, axis=1, keepdims=True)
        cand = jnp.where(dist == mval, iota_f, float(2 * _K))
        idx = jnp.min(cand, axis=1, keepdims=True).astype(jnp.int32)
        onehot = (iota == idx).astype(jnp.bfloat16)
        qhi = jax.lax.dot_general(
            onehot, hi, (((1,), (0,)), ((), ())), preferred_element_type=jnp.float32
        )
        qmid = jax.lax.dot_general(
            onehot, mid, (((1,), (0,)), ((), ())), preferred_element_type=jnp.float32
        )
        qlo = jax.lax.dot_general(
            onehot, lo, (((1,), (0,)), ((), ())), preferred_element_type=jnp.float32
        )
        quant = (qhi + qmid) + qlo
        diff = quant - resid
        loss_cols.append(jnp.sum(diff * diff, keepdims=True) * (1.0 / (_M * _D)))
        z_n = jnp.sqrt(r2)
        q_n = jnp.sqrt(_rowsum(quant * quant))
        z_norm = resid / (z_n + eps)
        q_norm = quant / (q_n + eps)
        s = z_norm + q_norm
        w = s / jnp.sqrt(_rowsum(s * s))
        ew = _rowsum(resid * w)
        ez = _rowsum(resid * z_norm)
        rotated = (resid - 2.0 * ew * w + 2.0 * ez * q_norm) * (q_n / (z_n + eps))
        resid = resid - rotated
        qout = qout + rotated
        idx_cols.append(idx)
    qout_ref[...] = qout
    idx_ref[...] = jnp.concatenate(idx_cols, axis=1)
    loss_ref[...] += jnp.concatenate(loss_cols, axis=1)


def kernel(x, codebooks):
    x2 = x.reshape(_M, _D)
    hi = jax.lax.optimization_barrier(codebooks.astype(jnp.bfloat16))
    r1 = jax.lax.optimization_barrier(codebooks - hi.astype(jnp.float32))
    mid = jax.lax.optimization_barrier(r1.astype(jnp.bfloat16))
    lo = jax.lax.optimization_barrier(
        (r1 - mid.astype(jnp.float32)).astype(jnp.bfloat16))
    qout, idx, loss = pl.pallas_call(
        _rvq_body,
        grid=(_M // _T,),
        in_specs=[
            pl.BlockSpec((_T, _D), lambda i: (i, 0)),
            pl.BlockSpec((_Q, _K, _D), lambda i: (0, 0, 0)),
            pl.BlockSpec((_Q, _K, _D), lambda i: (0, 0, 0)),
            pl.BlockSpec((_Q, _K, _D), lambda i: (0, 0, 0)),
            pl.BlockSpec((_Q, _K, _D), lambda i: (0, 0, 0)),
        ],
        out_specs=[
            pl.BlockSpec((_T, _D), lambda i: (i, 0)),
            pl.BlockSpec((_T, _Q), lambda i: (i, 0)),
            pl.BlockSpec((1, _Q), lambda i: (0, 0)),
        ],
        out_shape=[
            jax.ShapeDtypeStruct((_M, _D), jnp.float32),
            jax.ShapeDtypeStruct((_M, _Q), jnp.int32),
            jax.ShapeDtypeStruct((1, _Q), jnp.float32),
        ],
        scratch_shapes=[pltpu.VMEM((_Q, _K), jnp.float32),
                        pltpu.VMEM((_T, _D), jnp.float32),
                        pltpu.VMEM((1, _Q), jnp.float32)],
        compiler_params=pltpu.CompilerParams(
            dimension_semantics=("arbitrary",)
        ),
    )(x2, codebooks, hi, mid, lo)
    return (
        qout.reshape(_B, _N, _D),
        idx.reshape(_B, _N, _Q),
        loss.reshape(_Q),
    )
